# baseline (speedup 1.0000x reference)
import functools
import math

import jax
import jax.numpy as jnp
from jax import lax
from jax.experimental import pallas as pl
from jax.experimental.pallas import tpu as pltpu
from jax.experimental.pallas import tpu_sc as plsc

_N = 2048
_EMBED = 768
_HEADS = 12
_HEAD_DIM = _EMBED // _HEADS
_NUM_SAMPLES = 256
_SCALE = 1.0 / math.sqrt(float(_N))
_NEG = -1e30

_NW = 32
_ROWS_PER_W = _N // _NW
_CHUNK = 16
_G = _NUM_SAMPLES // 16
_NCHUNK = 4
_CW = _N // _NCHUNK
_CHUNK_SHIFT = _CW.bit_length() - 1


def _mask_body(samples_ref, mask_ref, cnt_ref, idx_v, buf_v, cnt_v):
    wid = lax.axis_index("s") * 2 + lax.axis_index("c")
    base = wid * _ROWS_PER_W
    pltpu.sync_copy(
        samples_ref.at[pl.ds(base * _NUM_SAMPLES, _ROWS_PER_W * _NUM_SAMPLES)],
        idx_v,
    )
    neg = jnp.full((16,), _NEG, jnp.float32)
    zero = jnp.zeros((16,), jnp.float32)

    def count(g, accs):
        col = lax.shift_right_logical(idx_v[pl.ds(g * 16, 16)], _CHUNK_SHIFT)
        return tuple(
            acc + plsc.all_reduce_population_count(col == cc)
            for cc, acc in enumerate(accs)
        )

    accs = lax.fori_loop(
        0,
        _ROWS_PER_W * _NUM_SAMPLES // 16,
        count,
        tuple(jnp.zeros((16,), jnp.int32) for _ in range(_NCHUNK)),
    )
    for cc in range(_NCHUNK):
        cnt_v[pl.ds(cc * 16, 16)] = accs[cc]
    pltpu.sync_copy(cnt_v, cnt_ref.at[wid])

    def fill(i, carry):
        for g in range(_N // 16):
            buf_v[i, pl.ds(g * 16, 16)] = neg
        return carry

    lax.fori_loop(0, _CHUNK, fill, 0)

    n_chunks = _ROWS_PER_W // _CHUNK
    for c in range(n_chunks):
        def scatter_row(r, carry, _c=c, _val=zero):
            row = jnp.full((16,), 0, jnp.int32) + r
            samp_off = (_c * _CHUNK + r) * _NUM_SAMPLES
            for g in range(_G):
                idx = idx_v[pl.ds(samp_off + g * 16, 16)]
                plsc.store_scatter(buf_v, [row, idx], _val)
            return carry

        lax.fori_loop(0, _CHUNK, scatter_row, 0)
        pltpu.sync_copy(
            buf_v, mask_ref.at[pl.ds(base + c * _CHUNK, _CHUNK)]
        )

        if c + 1 < n_chunks:
            lax.fori_loop(
                0, _CHUNK, functools.partial(scatter_row, _val=neg), 0
            )


@functools.cache
def _get_mask_builder():
    return pl.kernel(
        _mask_body,
        out_type=[
            jax.ShapeDtypeStruct((_N, _N), jnp.float32),
            jax.ShapeDtypeStruct((_NW, _NCHUNK * 16), jnp.int32),
        ],
        mesh=plsc.VectorSubcoreMesh(core_axis_name="c", subcore_axis_name="s"),
        scratch_types=[
            pltpu.VMEM((_ROWS_PER_W * _NUM_SAMPLES,), jnp.int32),
            pltpu.VMEM((_CHUNK, _N), jnp.float32),
            pltpu.VMEM((_NCHUNK * 16,), jnp.int32),
        ],
        compiler_params=pltpu.CompilerParams(
            needs_layout_passes=False, use_tc_tiling_on_sc=False
        ),
    )


def _build_mask(samples):
    return _get_mask_builder()(samples.reshape(-1))


_BR = 512
_NSPLIT = 2
_DN_T = (((1,), (1,)), ((), ()))
_DN_N = (((1,), (0,)), ((), ()))


def _attn_body(q_in_ref, wq_ref, bq_ref, key_ref, wk_ref, bk_ref,
               value_ref, wv_ref, bv_ref, mask_ref, cnt_ref, wo_ref, bo_ref,
               out_ref, k_s, v_s, kn_s, q_s, attn_s, av_s):
    r = pl.program_id(0)
    h = pl.program_id(1)

    @pl.when(r == 0)
    def _project_kv():
        k = lax.dot_general(
            key_ref[...], wk_ref[0], _DN_T, preferred_element_type=jnp.float32
        ) + bk_ref[0]
        kn_s[h, 0] = jnp.sqrt(jnp.max(jnp.sum(k * k, axis=1)))
        k_s[h] = k.astype(jnp.bfloat16)
        v = (
            lax.dot_general(
                value_ref[...], wv_ref[0], _DN_T,
                preferred_element_type=jnp.float32,
            ) + bv_ref[0]
        ).astype(jnp.bfloat16)
        col = lax.broadcasted_iota(jnp.int32, (_N, _HEAD_DIM), 1)
        pad = jnp.where(col == 0, 1.0, 0.0).astype(jnp.bfloat16)
        v_s[h] = jnp.concatenate([v, pad], axis=1)

    @pl.when(h == 0)
    def _project_q():
        q_all = lax.dot_general(
            q_in_ref[...], wq_ref[...], _DN_T,
            preferred_element_type=jnp.float32,
        ) + bq_ref[...]
        for h2 in range(_HEADS):
            q_s[h2] = q_all[:, h2 * _HEAD_DIM:(h2 + 1) * _HEAD_DIM].astype(
                jnp.bfloat16
            )

    qb = q_s[h]
    qf = qb.astype(jnp.float32)
    qn = jnp.sqrt(jnp.sum(qf * qf, axis=1, keepdims=True))
    m = qn * kn_s[h, 0]
    av_s[...] = jnp.zeros((_BR, 2 * _HEAD_DIM), jnp.float32)
    for cc in range(_NCHUNK):
        nz = cnt_ref[8 * r, cc * 16]
        for j in range(1, 8):
            nz = nz + cnt_ref[8 * r + j, cc * 16]

        @pl.when(nz > 0)
        def _chunk(cc=cc):
            logits = lax.dot_general(
                qb, k_s[h, cc * _CW:(cc + 1) * _CW], _DN_T,
                preferred_element_type=jnp.float32,
            )
            e = jnp.exp(
                logits - m + mask_ref[:, cc * _CW:(cc + 1) * _CW]
            ).astype(jnp.bfloat16)
            av_s[...] += lax.dot_general(
                e, v_s[h, cc * _CW:(cc + 1) * _CW], _DN_N,
                preferred_element_type=jnp.float32,
            )

    av = av_s[...]
    attn_s[h] = (
        av[:, :_HEAD_DIM] / av[:, _HEAD_DIM:_HEAD_DIM + 1]
    ).astype(jnp.bfloat16)

    @pl.when(h == _HEADS - 1)
    def _project_out():
        attn_all = jnp.concatenate(
            [attn_s[h2] for h2 in range(_HEADS)], axis=1
        )
        out_ref[...] = lax.dot_general(
            attn_all, wo_ref[...], _DN_T, preferred_element_type=jnp.float32
        ) + bo_ref[...]


def _head_spec():
    return pl.BlockSpec((1, _HEAD_DIM, _EMBED), lambda r, h: (h, 0, 0))


def _bias_spec():
    return pl.BlockSpec((1, 1, _HEAD_DIM), lambda r, h: (h, 0, 0))


_attn = pl.pallas_call(
    _attn_body,
    grid=(_N // _BR, _HEADS),
    in_specs=[
        pl.BlockSpec((_BR, _EMBED), lambda r, h: (r, 0)),
        pl.BlockSpec((_EMBED, _EMBED), lambda r, h: (0, 0)),
        pl.BlockSpec((1, _EMBED), lambda r, h: (0, 0)),
        pl.BlockSpec((_N, _EMBED), lambda r, h: (0, 0)),
        _head_spec(),
        _bias_spec(),
        pl.BlockSpec((_N, _EMBED), lambda r, h: (0, 0)),
        _head_spec(),
        _bias_spec(),
        pl.BlockSpec((_BR, _N), lambda r, h: (r, 0)),
        pl.BlockSpec(memory_space=pltpu.SMEM),
        pl.BlockSpec((_EMBED, _EMBED), lambda r, h: (0, 0)),
        pl.BlockSpec((1, _EMBED), lambda r, h: (0, 0)),
    ],
    out_specs=pl.BlockSpec((_BR, _EMBED), lambda r, h: (r, 0)),
    out_shape=jax.ShapeDtypeStruct((_N, _EMBED), jnp.float32),
    scratch_shapes=[
        pltpu.VMEM((_HEADS, _N, _HEAD_DIM), jnp.bfloat16),
        pltpu.VMEM((_HEADS, _N, 2 * _HEAD_DIM), jnp.bfloat16),
        pltpu.SMEM((_HEADS, 1), jnp.float32),
        pltpu.VMEM((_HEADS, _BR, _HEAD_DIM), jnp.bfloat16),
        pltpu.VMEM((_HEADS, _BR, _HEAD_DIM), jnp.bfloat16),
        pltpu.VMEM((_BR, 2 * _HEAD_DIM), jnp.float32),
    ],
    compiler_params=pltpu.CompilerParams(
        vmem_limit_bytes=100 * 1024 * 1024,
    ),
)


def kernel(query, key, value, Wq, bq, Wk, bk, Wv, bv, Wo, bo, samples):
    bf = jnp.bfloat16
    mask, cnt = _build_mask(samples)
    wk3 = Wk.reshape(_HEADS, _HEAD_DIM, _EMBED).astype(bf)
    wv3 = Wv.reshape(_HEADS, _HEAD_DIM, _EMBED).astype(bf)
    bk3 = bk.reshape(_HEADS, 1, _HEAD_DIM)
    bv3 = bv.reshape(_HEADS, 1, _HEAD_DIM)
    out = _attn(
        query.astype(bf), (Wq * _SCALE).astype(bf),
        (bq * _SCALE).reshape(1, _EMBED),
        key.astype(bf), wk3, bk3,
        value.astype(bf), wv3, bv3,
        mask, cnt, Wo.astype(bf), bo.reshape(1, _EMBED),
    )
    return out.reshape(_N, 1, _EMBED)


# speedup vs baseline: 1.0861x; 1.0861x over previous
import functools
import math

import jax
import jax.numpy as jnp
from jax import lax
from jax.experimental import pallas as pl
from jax.experimental.pallas import tpu as pltpu
from jax.experimental.pallas import tpu_sc as plsc

_N = 2048
_EMBED = 768
_HEADS = 12
_HEAD_DIM = _EMBED // _HEADS
_NUM_SAMPLES = 256
_SCALE = 1.0 / math.sqrt(float(_N))
_NEG = -1e30

_NW = 32
_ROWS_PER_W = _N // _NW
_CHUNK = 16
_G = _NUM_SAMPLES // 16
_NCHUNK = 8
_CW = _N // _NCHUNK
_CHUNK_SHIFT = _CW.bit_length() - 1
_WIN = 4 * _CW


def _mask_body(samples_ref, mask_ref, cnt_ref, idx_v, buf_v, cnt_v):
    wid = lax.axis_index("s") * 2 + lax.axis_index("c")
    base = wid * _ROWS_PER_W
    pltpu.sync_copy(
        samples_ref.at[pl.ds(base * _NUM_SAMPLES, _ROWS_PER_W * _NUM_SAMPLES)],
        idx_v,
    )
    neg = jnp.full((16,), _NEG, jnp.float32)
    zero = jnp.zeros((16,), jnp.float32)

    def count(g, accs):
        col = lax.shift_right_logical(idx_v[pl.ds(g * 16, 16)], _CHUNK_SHIFT)
        return tuple(
            acc + plsc.all_reduce_population_count(col == cc)
            for cc, acc in enumerate(accs)
        )

    accs = lax.fori_loop(
        0,
        _ROWS_PER_W * _NUM_SAMPLES // 16,
        count,
        tuple(jnp.zeros((16,), jnp.int32) for _ in range(_NCHUNK)),
    )
    for cc in range(_NCHUNK):
        cnt_v[pl.ds(cc * 16, 16)] = accs[cc]
    pltpu.sync_copy(cnt_v, cnt_ref.at[wid])

    def fill(i, carry):
        for g in range(_N // 16):
            buf_v[i, pl.ds(g * 16, 16)] = neg
        return carry

    lax.fori_loop(0, _CHUNK, fill, 0)

    n_chunks = _ROWS_PER_W // _CHUNK
    for c in range(n_chunks):
        def scatter_row(r, carry, _c=c, _val=zero):
            row = jnp.full((16,), 0, jnp.int32) + r
            samp_off = (_c * _CHUNK + r) * _NUM_SAMPLES
            for g in range(_G):
                idx = idx_v[pl.ds(samp_off + g * 16, 16)]
                plsc.store_scatter(buf_v, [row, idx], _val)
            return carry

        lax.fori_loop(0, _CHUNK, scatter_row, 0)
        pltpu.sync_copy(
            buf_v, mask_ref.at[pl.ds(base + c * _CHUNK, _CHUNK)]
        )

        if c + 1 < n_chunks:
            lax.fori_loop(
                0, _CHUNK, functools.partial(scatter_row, _val=neg), 0
            )


@functools.cache
def _get_mask_builder():
    return pl.kernel(
        _mask_body,
        out_type=[
            jax.ShapeDtypeStruct((_N, _N), jnp.float32),
            jax.ShapeDtypeStruct((_NW, _NCHUNK * 16), jnp.int32),
        ],
        mesh=plsc.VectorSubcoreMesh(core_axis_name="c", subcore_axis_name="s"),
        scratch_types=[
            pltpu.VMEM((_ROWS_PER_W * _NUM_SAMPLES,), jnp.int32),
            pltpu.VMEM((_CHUNK, _N), jnp.float32),
            pltpu.VMEM((_NCHUNK * 16,), jnp.int32),
        ],
        compiler_params=pltpu.CompilerParams(
            needs_layout_passes=False, use_tc_tiling_on_sc=False
        ),
    )


def _build_mask(samples):
    return _get_mask_builder()(samples.reshape(-1))


_BR = 512
_NSPLIT = 2
_DN_T = (((1,), (1,)), ((), ()))
_DN_N = (((1,), (0,)), ((), ()))


def _attn_body(q_in_ref, wq_ref, bq_ref, key_ref, wk_ref, bk_ref,
               value_ref, wv_ref, bv_ref, mask_ref, fast_ref, start_ref,
               wo_ref, bo_ref, out_ref, k_s, v_s, kn_s, q_s, attn_s, av_s):
    r = pl.program_id(0)
    h = pl.program_id(1)

    @pl.when(r == 0)
    def _project_kv():
        k = lax.dot_general(
            key_ref[...], wk_ref[0], _DN_T, preferred_element_type=jnp.float32
        ) + bk_ref[0]
        kn_s[h, 0] = jnp.sqrt(jnp.max(jnp.sum(k * k, axis=1)))
        k_s[h] = k.astype(jnp.bfloat16)
        v = (
            lax.dot_general(
                value_ref[...], wv_ref[0], _DN_T,
                preferred_element_type=jnp.float32,
            ) + bv_ref[0]
        ).astype(jnp.bfloat16)
        col = lax.broadcasted_iota(jnp.int32, (_N, _HEAD_DIM), 1)
        pad = jnp.where(col == 0, 1.0, 0.0).astype(jnp.bfloat16)
        v_s[h] = jnp.concatenate([v, pad], axis=1)

    @pl.when(h == 0)
    def _project_q():
        q_all = lax.dot_general(
            q_in_ref[...], wq_ref[...], _DN_T,
            preferred_element_type=jnp.float32,
        ) + bq_ref[...]
        for h2 in range(_HEADS):
            q_s[h2] = q_all[:, h2 * _HEAD_DIM:(h2 + 1) * _HEAD_DIM].astype(
                jnp.bfloat16
            )

    qb = q_s[h]
    qf = qb.astype(jnp.float32)
    qn = jnp.sqrt(jnp.sum(qf * qf, axis=1, keepdims=True))
    m = qn * kn_s[h, 0]

    @pl.when(fast_ref[r] == 1)
    def _windowed():
        st = pl.multiple_of(start_ref[r], _CW)
        logits = lax.dot_general(
            qb, k_s[h, pl.ds(st, _WIN)], _DN_T,
            preferred_element_type=jnp.float32,
        )
        e = jnp.exp(logits - m + mask_ref[:, pl.ds(st, _WIN)]).astype(
            jnp.bfloat16
        )
        av_s[...] = lax.dot_general(
            e, v_s[h, pl.ds(st, _WIN)], _DN_N,
            preferred_element_type=jnp.float32,
        )

    @pl.when(fast_ref[r] == 0)
    def _full():
        logits = lax.dot_general(
            qb, k_s[h], _DN_T, preferred_element_type=jnp.float32
        )
        e = jnp.exp(logits - m + mask_ref[...]).astype(jnp.bfloat16)
        av_s[...] = lax.dot_general(
            e, v_s[h], _DN_N, preferred_element_type=jnp.float32
        )

    av = av_s[...]
    attn_s[h] = (
        av[:, :_HEAD_DIM] / av[:, _HEAD_DIM:_HEAD_DIM + 1]
    ).astype(jnp.bfloat16)

    @pl.when(h == _HEADS - 1)
    def _project_out():
        attn_all = jnp.concatenate(
            [attn_s[h2] for h2 in range(_HEADS)], axis=1
        )
        out_ref[...] = lax.dot_general(
            attn_all, wo_ref[...], _DN_T, preferred_element_type=jnp.float32
        ) + bo_ref[...]


def _head_spec():
    return pl.BlockSpec((1, _HEAD_DIM, _EMBED), lambda r, h: (h, 0, 0))


def _bias_spec():
    return pl.BlockSpec((1, 1, _HEAD_DIM), lambda r, h: (h, 0, 0))


_attn = pl.pallas_call(
    _attn_body,
    grid=(_N // _BR, _HEADS),
    in_specs=[
        pl.BlockSpec((_BR, _EMBED), lambda r, h: (r, 0)),
        pl.BlockSpec((_EMBED, _EMBED), lambda r, h: (0, 0)),
        pl.BlockSpec((1, _EMBED), lambda r, h: (0, 0)),
        pl.BlockSpec((_N, _EMBED), lambda r, h: (0, 0)),
        _head_spec(),
        _bias_spec(),
        pl.BlockSpec((_N, _EMBED), lambda r, h: (0, 0)),
        _head_spec(),
        _bias_spec(),
        pl.BlockSpec((_BR, _N), lambda r, h: (r, 0)),
        pl.BlockSpec(memory_space=pltpu.SMEM),
        pl.BlockSpec(memory_space=pltpu.SMEM),
        pl.BlockSpec((_EMBED, _EMBED), lambda r, h: (0, 0)),
        pl.BlockSpec((1, _EMBED), lambda r, h: (0, 0)),
    ],
    out_specs=pl.BlockSpec((_BR, _EMBED), lambda r, h: (r, 0)),
    out_shape=jax.ShapeDtypeStruct((_N, _EMBED), jnp.float32),
    scratch_shapes=[
        pltpu.VMEM((_HEADS, _N, _HEAD_DIM), jnp.bfloat16),
        pltpu.VMEM((_HEADS, _N, 2 * _HEAD_DIM), jnp.bfloat16),
        pltpu.SMEM((_HEADS, 1), jnp.float32),
        pltpu.VMEM((_HEADS, _BR, _HEAD_DIM), jnp.bfloat16),
        pltpu.VMEM((_HEADS, _BR, _HEAD_DIM), jnp.bfloat16),
        pltpu.VMEM((_BR, 2 * _HEAD_DIM), jnp.float32),
    ],
    compiler_params=pltpu.CompilerParams(
        vmem_limit_bytes=100 * 1024 * 1024,
    ),
)


def kernel(query, key, value, Wq, bq, Wk, bk, Wv, bv, Wo, bo, samples):
    bf = jnp.bfloat16
    mask, cnt = _build_mask(samples)
    nz = cnt.reshape(_N // _BR, _BR // _ROWS_PER_W, _NCHUNK, 16)[..., 0].sum(1)
    total = nz.sum(1, keepdims=True)
    npos = _NCHUNK - _WIN // _CW + 1
    covered = jnp.stack(
        [nz[:, p:p + _WIN // _CW].sum(1) for p in range(npos)], axis=1
    )
    fits = covered == total
    fast = fits.any(1).astype(jnp.int32)
    start = (jnp.argmax(fits, axis=1) * _CW).astype(jnp.int32)
    wk3 = Wk.reshape(_HEADS, _HEAD_DIM, _EMBED).astype(bf)
    wv3 = Wv.reshape(_HEADS, _HEAD_DIM, _EMBED).astype(bf)
    bk3 = bk.reshape(_HEADS, 1, _HEAD_DIM)
    bv3 = bv.reshape(_HEADS, 1, _HEAD_DIM)
    out = _attn(
        query.astype(bf), (Wq * _SCALE).astype(bf),
        (bq * _SCALE).reshape(1, _EMBED),
        key.astype(bf), wk3, bk3,
        value.astype(bf), wv3, bv3,
        mask, fast, start, Wo.astype(bf), bo.reshape(1, _EMBED),
    )
    return out.reshape(_N, 1, _EMBED)


# speedup vs baseline: 1.1085x; 1.0206x over previous
import functools
import math

import jax
import jax.numpy as jnp
from jax import lax
from jax.experimental import pallas as pl
from jax.experimental.pallas import tpu as pltpu
from jax.experimental.pallas import tpu_sc as plsc

_N = 2048
_EMBED = 768
_HEADS = 12
_HEAD_DIM = _EMBED // _HEADS
_NUM_SAMPLES = 256
_SCALE = 1.0 / math.sqrt(float(_N))
_NEG = -1e30

_NW = 32
_ROWS_PER_W = _N // _NW
_CHUNK = 16
_G = _NUM_SAMPLES // 16
_NCHUNK = 8
_CW = _N // _NCHUNK
_CHUNK_SHIFT = _CW.bit_length() - 1
_WIN = 4 * _CW


def _mask_body(samples_ref, mask_ref, cnt_ref, idx_v, buf_v, cnt_v):
    wid = lax.axis_index("s") * 2 + lax.axis_index("c")
    base = wid * _ROWS_PER_W
    pltpu.sync_copy(
        samples_ref.at[pl.ds(base * _NUM_SAMPLES, _ROWS_PER_W * _NUM_SAMPLES)],
        idx_v,
    )
    neg = jnp.full((16,), _NEG, jnp.float32)
    zero = jnp.zeros((16,), jnp.float32)

    def count(g, accs):
        col = lax.shift_right_logical(idx_v[pl.ds(g * 16, 16)], _CHUNK_SHIFT)
        return tuple(
            acc + plsc.all_reduce_population_count(col == cc)
            for cc, acc in enumerate(accs)
        )

    accs = lax.fori_loop(
        0,
        _ROWS_PER_W * _NUM_SAMPLES // 16,
        count,
        tuple(jnp.zeros((16,), jnp.int32) for _ in range(_NCHUNK)),
    )
    for cc in range(_NCHUNK):
        cnt_v[pl.ds(cc * 16, 16)] = accs[cc]
    pltpu.sync_copy(cnt_v, cnt_ref.at[wid])

    def fill(i, carry):
        for g in range(_N // 16):
            buf_v[i, pl.ds(g * 16, 16)] = neg
        return carry

    lax.fori_loop(0, _CHUNK, fill, 0)

    n_chunks = _ROWS_PER_W // _CHUNK
    for c in range(n_chunks):
        def scatter_row(r, carry, _c=c, _val=zero):
            row = jnp.full((16,), 0, jnp.int32) + r
            samp_off = (_c * _CHUNK + r) * _NUM_SAMPLES
            for g in range(_G):
                idx = idx_v[pl.ds(samp_off + g * 16, 16)]
                plsc.store_scatter(buf_v, [row, idx], _val)
            return carry

        lax.fori_loop(0, _CHUNK, scatter_row, 0)
        pltpu.sync_copy(
            buf_v, mask_ref.at[pl.ds(base + c * _CHUNK, _CHUNK)]
        )

        if c + 1 < n_chunks:
            lax.fori_loop(
                0, _CHUNK, functools.partial(scatter_row, _val=neg), 0
            )


@functools.cache
def _get_mask_builder():
    return pl.kernel(
        _mask_body,
        out_type=[
            jax.ShapeDtypeStruct((_N, _N), jnp.float32),
            jax.ShapeDtypeStruct((_NW, _NCHUNK * 16), jnp.int32),
        ],
        mesh=plsc.VectorSubcoreMesh(core_axis_name="c", subcore_axis_name="s"),
        scratch_types=[
            pltpu.VMEM((_ROWS_PER_W * _NUM_SAMPLES,), jnp.int32),
            pltpu.VMEM((_CHUNK, _N), jnp.float32),
            pltpu.VMEM((_NCHUNK * 16,), jnp.int32),
        ],
        compiler_params=pltpu.CompilerParams(
            needs_layout_passes=False, use_tc_tiling_on_sc=False
        ),
    )


def _build_mask(samples):
    return _get_mask_builder()(samples.reshape(-1))


_BR = 512
_NSPLIT = 2
_DN_T = (((1,), (1,)), ((), ()))
_DN_N = (((1,), (0,)), ((), ()))


def _attn_body(q_in_ref, wq_ref, bq_ref, key_ref, wk_ref, bk_ref,
               value_ref, wv_ref, bv_ref, mask_ref, fast_ref, start_ref,
               wo_ref, bo_ref, out_ref, k_s, v_s, kn_s, q_s, attn_s, av_s):
    r = pl.program_id(0)
    h = pl.program_id(1)

    @pl.when(r == 0)
    def _project_kv():
        k = lax.dot_general(
            key_ref[...], wk_ref[0], _DN_T, preferred_element_type=jnp.float32
        ) + bk_ref[0]
        kn_s[h, 0] = jnp.sqrt(jnp.max(jnp.sum(k * k, axis=1)))
        k_s[h] = k.astype(jnp.bfloat16)
        v = (
            lax.dot_general(
                value_ref[...], wv_ref[0], _DN_T,
                preferred_element_type=jnp.float32,
            ) + bv_ref[0]
        ).astype(jnp.bfloat16)
        col = lax.broadcasted_iota(jnp.int32, (_N, _HEAD_DIM), 1)
        pad = jnp.where(col == 0, 1.0, 0.0).astype(jnp.bfloat16)
        v_s[h] = jnp.concatenate([v, pad], axis=1)

    @pl.when(h == 0)
    def _project_q():
        q_all = lax.dot_general(
            q_in_ref[...], wq_ref[...], _DN_T,
            preferred_element_type=jnp.float32,
        ) + bq_ref[...]
        for h2 in range(_HEADS):
            q_s[h2] = q_all[:, h2 * _HEAD_DIM:(h2 + 1) * _HEAD_DIM].astype(
                jnp.bfloat16
            )

    qb = q_s[h]
    qf = qb.astype(jnp.float32)
    qn = jnp.sqrt(jnp.sum(qf * qf, axis=1, keepdims=True))
    m = qn * kn_s[h, 0]

    @pl.when(fast_ref[r] == 1)
    def _windowed():
        st = pl.multiple_of(start_ref[r], _CW)
        logits = lax.dot_general(
            qb, k_s[h, pl.ds(st, _WIN)], _DN_T,
            preferred_element_type=jnp.float32,
        )
        e = jnp.exp(
            logits - m + mask_ref[:, pl.ds(st, _WIN)].astype(jnp.float32)
        ).astype(jnp.bfloat16)
        av_s[...] = lax.dot_general(
            e, v_s[h, pl.ds(st, _WIN)], _DN_N,
            preferred_element_type=jnp.float32,
        )

    @pl.when(fast_ref[r] == 0)
    def _full():
        logits = lax.dot_general(
            qb, k_s[h], _DN_T, preferred_element_type=jnp.float32
        )
        e = jnp.exp(
            logits - m + mask_ref[...].astype(jnp.float32)
        ).astype(jnp.bfloat16)
        av_s[...] = lax.dot_general(
            e, v_s[h], _DN_N, preferred_element_type=jnp.float32
        )

    av = av_s[...]
    attn_s[h] = (
        av[:, :_HEAD_DIM] / av[:, _HEAD_DIM:_HEAD_DIM + 1]
    ).astype(jnp.bfloat16)

    @pl.when(h == _HEADS - 1)
    def _project_out():
        attn_all = jnp.concatenate(
            [attn_s[h2] for h2 in range(_HEADS)], axis=1
        )
        out_ref[...] = lax.dot_general(
            attn_all, wo_ref[...], _DN_T, preferred_element_type=jnp.float32
        ) + bo_ref[...]


def _head_spec():
    return pl.BlockSpec((1, _HEAD_DIM, _EMBED), lambda r, h: (h, 0, 0))


def _bias_spec():
    return pl.BlockSpec((1, 1, _HEAD_DIM), lambda r, h: (h, 0, 0))


_attn = pl.pallas_call(
    _attn_body,
    grid=(_N // _BR, _HEADS),
    in_specs=[
        pl.BlockSpec((_BR, _EMBED), lambda r, h: (r, 0)),
        pl.BlockSpec((_EMBED, _EMBED), lambda r, h: (0, 0)),
        pl.BlockSpec((1, _EMBED), lambda r, h: (0, 0)),
        pl.BlockSpec((_N, _EMBED), lambda r, h: (0, 0)),
        _head_spec(),
        _bias_spec(),
        pl.BlockSpec((_N, _EMBED), lambda r, h: (0, 0)),
        _head_spec(),
        _bias_spec(),
        pl.BlockSpec((_BR, _N), lambda r, h: (r, 0)),
        pl.BlockSpec(memory_space=pltpu.SMEM),
        pl.BlockSpec(memory_space=pltpu.SMEM),
        pl.BlockSpec((_EMBED, _EMBED), lambda r, h: (0, 0)),
        pl.BlockSpec((1, _EMBED), lambda r, h: (0, 0)),
    ],
    out_specs=pl.BlockSpec((_BR, _EMBED), lambda r, h: (r, 0)),
    out_shape=jax.ShapeDtypeStruct((_N, _EMBED), jnp.float32),
    scratch_shapes=[
        pltpu.VMEM((_HEADS, _N, _HEAD_DIM), jnp.bfloat16),
        pltpu.VMEM((_HEADS, _N, 2 * _HEAD_DIM), jnp.bfloat16),
        pltpu.SMEM((_HEADS, 1), jnp.float32),
        pltpu.VMEM((_HEADS, _BR, _HEAD_DIM), jnp.bfloat16),
        pltpu.VMEM((_HEADS, _BR, _HEAD_DIM), jnp.bfloat16),
        pltpu.VMEM((_BR, 2 * _HEAD_DIM), jnp.float32),
    ],
    compiler_params=pltpu.CompilerParams(
        vmem_limit_bytes=100 * 1024 * 1024,
    ),
)


def kernel(query, key, value, Wq, bq, Wk, bk, Wv, bv, Wo, bo, samples):
    bf = jnp.bfloat16
    mask, cnt = _build_mask(samples)
    nz = cnt.reshape(_N // _BR, _BR // _ROWS_PER_W, _NCHUNK, 16)[..., 0].sum(1)
    total = nz.sum(1, keepdims=True)
    npos = _NCHUNK - _WIN // _CW + 1
    covered = jnp.stack(
        [nz[:, p:p + _WIN // _CW].sum(1) for p in range(npos)], axis=1
    )
    fits = covered == total
    fast = fits.any(1).astype(jnp.int32)
    start = (jnp.argmax(fits, axis=1) * _CW).astype(jnp.int32)
    wk3 = Wk.reshape(_HEADS, _HEAD_DIM, _EMBED).astype(bf)
    wv3 = Wv.reshape(_HEADS, _HEAD_DIM, _EMBED).astype(bf)
    bk3 = bk.reshape(_HEADS, 1, _HEAD_DIM)
    bv3 = bv.reshape(_HEADS, 1, _HEAD_DIM)
    out = _attn(
        query.astype(bf), (Wq * _SCALE).astype(bf),
        (bq * _SCALE).reshape(1, _EMBED),
        key.astype(bf), wk3, bk3,
        value.astype(bf), wv3, bv3,
        mask.astype(bf), fast, start, Wo.astype(bf), bo.reshape(1, _EMBED),
    )
    return out.reshape(_N, 1, _EMBED)


# speedup vs baseline: 1.1250x; 1.0149x over previous
import functools
import math

import jax
import jax.numpy as jnp
from jax import lax
from jax.experimental import pallas as pl
from jax.experimental.pallas import tpu as pltpu
from jax.experimental.pallas import tpu_sc as plsc

_N = 2048
_EMBED = 768
_HEADS = 12
_HEAD_DIM = _EMBED // _HEADS
_NUM_SAMPLES = 256
_SCALE = 1.0 / math.sqrt(float(_N))
_NEG = -1e30

_NW = 32
_ROWS_PER_W = _N // _NW
_CHUNK = 16
_G = _NUM_SAMPLES // 16
_NCHUNK = 8
_CW = _N // _NCHUNK
_CHUNK_SHIFT = _CW.bit_length() - 1
_WIN = 4 * _CW


def _mask_body(samples_ref, mask_ref, cnt_ref, idx_v, buf_v, cnt_v, dma_sems):
    wid = lax.axis_index("s") * 2 + lax.axis_index("c")
    base = wid * _ROWS_PER_W
    pltpu.sync_copy(
        samples_ref.at[pl.ds(base * _NUM_SAMPLES, _ROWS_PER_W * _NUM_SAMPLES)],
        idx_v,
    )
    neg = jnp.full((16,), _NEG, jnp.float32)
    zero = jnp.zeros((16,), jnp.float32)

    def count(g, accs):
        col = lax.shift_right_logical(idx_v[pl.ds(g * 16, 16)], _CHUNK_SHIFT)
        return tuple(
            acc + plsc.all_reduce_population_count(col == cc)
            for cc, acc in enumerate(accs)
        )

    accs = lax.fori_loop(
        0,
        _ROWS_PER_W * _NUM_SAMPLES // 16,
        count,
        tuple(jnp.zeros((16,), jnp.int32) for _ in range(_NCHUNK)),
    )
    for cc in range(_NCHUNK):
        cnt_v[pl.ds(cc * 16, 16)] = accs[cc]
    pltpu.sync_copy(cnt_v, cnt_ref.at[wid])

    def fill(i, carry):
        for g in range(_N // 16):
            buf_v[0, i, pl.ds(g * 16, 16)] = neg
            buf_v[1, i, pl.ds(g * 16, 16)] = neg
        return carry

    lax.fori_loop(0, _CHUNK, fill, 0)

    def scatter_chunk(c, b, val):
        def scatter_row(r, carry):
            row = jnp.full((16,), 0, jnp.int32) + r
            samp_off = (c * _CHUNK + r) * _NUM_SAMPLES
            for g in range(_G):
                idx = idx_v[pl.ds(samp_off + g * 16, 16)]
                plsc.store_scatter(buf_v.at[b], [row, idx], val)
            return carry

        lax.fori_loop(0, _CHUNK, scatter_row, 0)

    n_chunks = _ROWS_PER_W // _CHUNK
    handles = [None, None]
    for c in range(n_chunks):
        b = c % 2
        if handles[b] is not None:
            handles[b].wait()
            scatter_chunk(c - 2, b, neg)
        scatter_chunk(c, b, zero)
        handles[b] = pltpu.make_async_copy(
            buf_v.at[b],
            mask_ref.at[pl.ds(base + c * _CHUNK, _CHUNK)],
            dma_sems.at[b],
        )
        handles[b].start()
    handles[0].wait()
    handles[1].wait()


@functools.cache
def _get_mask_builder():
    return pl.kernel(
        _mask_body,
        out_type=[
            jax.ShapeDtypeStruct((_N, _N), jnp.float32),
            jax.ShapeDtypeStruct((_NW, _NCHUNK * 16), jnp.int32),
        ],
        mesh=plsc.VectorSubcoreMesh(core_axis_name="c", subcore_axis_name="s"),
        scratch_types=[
            pltpu.VMEM((_ROWS_PER_W * _NUM_SAMPLES,), jnp.int32),
            pltpu.VMEM((2, _CHUNK, _N), jnp.float32),
            pltpu.VMEM((_NCHUNK * 16,), jnp.int32),
            pltpu.SemaphoreType.DMA((2,)),
        ],
        compiler_params=pltpu.CompilerParams(
            needs_layout_passes=False, use_tc_tiling_on_sc=False
        ),
    )


def _build_mask(samples):
    return _get_mask_builder()(samples.reshape(-1))


_BR = 512
_NSPLIT = 2
_DN_T = (((1,), (1,)), ((), ()))
_DN_N = (((1,), (0,)), ((), ()))


def _attn_body(q_in_ref, wq_ref, bq_ref, key_ref, wk_ref, bk_ref,
               value_ref, wv_ref, bv_ref, mask_ref, fast_ref, start_ref,
               wo_ref, bo_ref, out_ref, k_s, v_s, kn_s, q_s, attn_s, av_s):
    r = pl.program_id(0)
    h = pl.program_id(1)

    @pl.when(r == 0)
    def _project_kv():
        k = lax.dot_general(
            key_ref[...], wk_ref[0], _DN_T, preferred_element_type=jnp.float32
        ) + bk_ref[0]
        kn_s[h, 0] = jnp.sqrt(jnp.max(jnp.sum(k * k, axis=1)))
        k_s[h] = k.astype(jnp.bfloat16)
        v = (
            lax.dot_general(
                value_ref[...], wv_ref[0], _DN_T,
                preferred_element_type=jnp.float32,
            ) + bv_ref[0]
        ).astype(jnp.bfloat16)
        col = lax.broadcasted_iota(jnp.int32, (_N, _HEAD_DIM), 1)
        pad = jnp.where(col == 0, 1.0, 0.0).astype(jnp.bfloat16)
        v_s[h] = jnp.concatenate([v, pad], axis=1)

    @pl.when(h == 0)
    def _project_q():
        q_all = lax.dot_general(
            q_in_ref[...], wq_ref[...], _DN_T,
            preferred_element_type=jnp.float32,
        ) + bq_ref[...]
        for h2 in range(_HEADS):
            q_s[h2] = q_all[:, h2 * _HEAD_DIM:(h2 + 1) * _HEAD_DIM].astype(
                jnp.bfloat16
            )

    qb = q_s[h]
    qf = qb.astype(jnp.float32)
    qn = jnp.sqrt(jnp.sum(qf * qf, axis=1, keepdims=True))
    m = qn * kn_s[h, 0]

    @pl.when(fast_ref[r] == 1)
    def _windowed():
        st = pl.multiple_of(start_ref[r], _CW)
        logits = lax.dot_general(
            qb, k_s[h, pl.ds(st, _WIN)], _DN_T,
            preferred_element_type=jnp.float32,
        )
        e = jnp.exp(
            logits - m + mask_ref[:, pl.ds(st, _WIN)].astype(jnp.float32)
        ).astype(jnp.bfloat16)
        av_s[...] = lax.dot_general(
            e, v_s[h, pl.ds(st, _WIN)], _DN_N,
            preferred_element_type=jnp.float32,
        )

    @pl.when(fast_ref[r] == 0)
    def _full():
        logits = lax.dot_general(
            qb, k_s[h], _DN_T, preferred_element_type=jnp.float32
        )
        e = jnp.exp(
            logits - m + mask_ref[...].astype(jnp.float32)
        ).astype(jnp.bfloat16)
        av_s[...] = lax.dot_general(
            e, v_s[h], _DN_N, preferred_element_type=jnp.float32
        )

    av = av_s[...]
    attn_s[h] = (
        av[:, :_HEAD_DIM] / av[:, _HEAD_DIM:_HEAD_DIM + 1]
    ).astype(jnp.bfloat16)

    @pl.when(h == _HEADS - 1)
    def _project_out():
        attn_all = jnp.concatenate(
            [attn_s[h2] for h2 in range(_HEADS)], axis=1
        )
        out_ref[...] = lax.dot_general(
            attn_all, wo_ref[...], _DN_T, preferred_element_type=jnp.float32
        ) + bo_ref[...]


def _head_spec():
    return pl.BlockSpec((1, _HEAD_DIM, _EMBED), lambda r, h: (h, 0, 0))


def _bias_spec():
    return pl.BlockSpec((1, 1, _HEAD_DIM), lambda r, h: (h, 0, 0))


_attn = pl.pallas_call(
    _attn_body,
    grid=(_N // _BR, _HEADS),
    in_specs=[
        pl.BlockSpec((_BR, _EMBED), lambda r, h: (r, 0)),
        pl.BlockSpec((_EMBED, _EMBED), lambda r, h: (0, 0)),
        pl.BlockSpec((1, _EMBED), lambda r, h: (0, 0)),
        pl.BlockSpec((_N, _EMBED), lambda r, h: (0, 0)),
        _head_spec(),
        _bias_spec(),
        pl.BlockSpec((_N, _EMBED), lambda r, h: (0, 0)),
        _head_spec(),
        _bias_spec(),
        pl.BlockSpec((_BR, _N), lambda r, h: (r, 0)),
        pl.BlockSpec(memory_space=pltpu.SMEM),
        pl.BlockSpec(memory_space=pltpu.SMEM),
        pl.BlockSpec((_EMBED, _EMBED), lambda r, h: (0, 0)),
        pl.BlockSpec((1, _EMBED), lambda r, h: (0, 0)),
    ],
    out_specs=pl.BlockSpec((_BR, _EMBED), lambda r, h: (r, 0)),
    out_shape=jax.ShapeDtypeStruct((_N, _EMBED), jnp.float32),
    scratch_shapes=[
        pltpu.VMEM((_HEADS, _N, _HEAD_DIM), jnp.bfloat16),
        pltpu.VMEM((_HEADS, _N, 2 * _HEAD_DIM), jnp.bfloat16),
        pltpu.SMEM((_HEADS, 1), jnp.float32),
        pltpu.VMEM((_HEADS, _BR, _HEAD_DIM), jnp.bfloat16),
        pltpu.VMEM((_HEADS, _BR, _HEAD_DIM), jnp.bfloat16),
        pltpu.VMEM((_BR, 2 * _HEAD_DIM), jnp.float32),
    ],
    compiler_params=pltpu.CompilerParams(
        vmem_limit_bytes=100 * 1024 * 1024,
    ),
)


def kernel(query, key, value, Wq, bq, Wk, bk, Wv, bv, Wo, bo, samples):
    bf = jnp.bfloat16
    mask, cnt = _build_mask(samples)
    nz = cnt.reshape(_N // _BR, _BR // _ROWS_PER_W, _NCHUNK, 16)[..., 0].sum(1)
    total = nz.sum(1, keepdims=True)
    npos = _NCHUNK - _WIN // _CW + 1
    covered = jnp.stack(
        [nz[:, p:p + _WIN // _CW].sum(1) for p in range(npos)], axis=1
    )
    fits = covered == total
    fast = fits.any(1).astype(jnp.int32)
    start = (jnp.argmax(fits, axis=1) * _CW).astype(jnp.int32)
    wk3 = Wk.reshape(_HEADS, _HEAD_DIM, _EMBED).astype(bf)
    wv3 = Wv.reshape(_HEADS, _HEAD_DIM, _EMBED).astype(bf)
    bk3 = bk.reshape(_HEADS, 1, _HEAD_DIM)
    bv3 = bv.reshape(_HEADS, 1, _HEAD_DIM)
    out = _attn(
        query.astype(bf), (Wq * _SCALE).astype(bf),
        (bq * _SCALE).reshape(1, _EMBED),
        key.astype(bf), wk3, bk3,
        value.astype(bf), wv3, bv3,
        mask.astype(bf), fast, start, Wo.astype(bf), bo.reshape(1, _EMBED),
    )
    return out.reshape(_N, 1, _EMBED)


# speedup vs baseline: 1.1769x; 1.0462x over previous
import functools
import math

import jax
import jax.numpy as jnp
from jax import lax
from jax.experimental import pallas as pl
from jax.experimental.pallas import tpu as pltpu
from jax.experimental.pallas import tpu_sc as plsc

_N = 2048
_EMBED = 768
_HEADS = 12
_HEAD_DIM = _EMBED // _HEADS
_NUM_SAMPLES = 256
_SCALE = 1.0 / math.sqrt(float(_N))
_NEG = -1e30

_NW = 32
_ROWS_PER_W = _N // _NW
_CHUNK = 16
_G = _NUM_SAMPLES // 16
_NCHUNK = 8
_CW = _N // _NCHUNK
_CHUNK_SHIFT = _CW.bit_length() - 1
_WIN = 4 * _CW


def _mask_body(samples_ref, mask_ref, cnt_ref, idx_v, buf_v, cnt_v, dma_sems):
    wid = lax.axis_index("s") * 2 + lax.axis_index("c")
    base = wid * _ROWS_PER_W
    pltpu.sync_copy(
        samples_ref.at[pl.ds(base * _NUM_SAMPLES, _ROWS_PER_W * _NUM_SAMPLES)],
        idx_v,
    )
    neg = jnp.full((16,), _NEG, jnp.float32)
    zero = jnp.zeros((16,), jnp.float32)

    def count(g, accs):
        col = lax.shift_right_logical(idx_v[pl.ds(g * 16, 16)], _CHUNK_SHIFT)
        return tuple(
            acc + plsc.all_reduce_population_count(col == cc)
            for cc, acc in enumerate(accs)
        )

    accs = lax.fori_loop(
        0,
        _ROWS_PER_W * _NUM_SAMPLES // 16,
        count,
        tuple(jnp.zeros((16,), jnp.int32) for _ in range(_NCHUNK)),
    )
    for cc in range(_NCHUNK):
        cnt_v[pl.ds(cc * 16, 16)] = accs[cc]
    pltpu.sync_copy(cnt_v, cnt_ref.at[wid])

    def fill(i, carry):
        for g in range(_N // 16):
            buf_v[0, i, pl.ds(g * 16, 16)] = neg
            buf_v[1, i, pl.ds(g * 16, 16)] = neg
        return carry

    lax.fori_loop(0, _CHUNK, fill, 0)

    def scatter_chunk(c, b, val):
        def scatter_row(r, carry):
            row = jnp.full((16,), 0, jnp.int32) + r
            samp_off = (c * _CHUNK + r) * _NUM_SAMPLES
            for g in range(_G):
                idx = idx_v[pl.ds(samp_off + g * 16, 16)]
                plsc.store_scatter(buf_v.at[b], [row, idx], val)
            return carry

        lax.fori_loop(0, _CHUNK, scatter_row, 0)

    n_chunks = _ROWS_PER_W // _CHUNK
    handles = [None, None]
    for c in range(n_chunks):
        b = c % 2
        if handles[b] is not None:
            handles[b].wait()
            scatter_chunk(c - 2, b, neg)
        scatter_chunk(c, b, zero)
        handles[b] = pltpu.make_async_copy(
            buf_v.at[b],
            mask_ref.at[pl.ds(base + c * _CHUNK, _CHUNK)],
            dma_sems.at[b],
        )
        handles[b].start()
    handles[0].wait()
    handles[1].wait()


@functools.cache
def _get_mask_builder():
    return pl.kernel(
        _mask_body,
        out_type=[
            jax.ShapeDtypeStruct((_N, _N), jnp.float32),
            jax.ShapeDtypeStruct((_NW, _NCHUNK * 16), jnp.int32),
        ],
        mesh=plsc.VectorSubcoreMesh(core_axis_name="c", subcore_axis_name="s"),
        scratch_types=[
            pltpu.VMEM((_ROWS_PER_W * _NUM_SAMPLES,), jnp.int32),
            pltpu.VMEM((2, _CHUNK, _N), jnp.float32),
            pltpu.VMEM((_NCHUNK * 16,), jnp.int32),
            pltpu.SemaphoreType.DMA((2,)),
        ],
        compiler_params=pltpu.CompilerParams(
            needs_layout_passes=False, use_tc_tiling_on_sc=False
        ),
    )


def _build_mask(samples):
    return _get_mask_builder()(samples.reshape(-1))


_BR = 512
_NSPLIT = 2
_DN_T = (((1,), (1,)), ((), ()))
_DN_N = (((1,), (0,)), ((), ()))


def _attn_body(q_in_ref, wq_ref, bq_ref, key_ref, wk_ref, bk_ref,
               value_ref, wv_ref, bv_ref, mask_ref, fast_ref, start_ref,
               wo_ref, bo_ref, out_ref, k_s, v_s, kn_s, q_s, attn_s, av_s):
    r = pl.program_id(0)
    h = pl.program_id(1)

    @pl.when(jnp.logical_and(r == 0, h == 0))
    def _project_kv():
        k_all = lax.dot_general(
            key_ref[...], wk_ref[...], _DN_T,
            preferred_element_type=jnp.float32,
        ) + bk_ref[...]
        v_all = lax.dot_general(
            value_ref[...], wv_ref[...], _DN_T,
            preferred_element_type=jnp.float32,
        ) + bv_ref[...]
        col = lax.broadcasted_iota(jnp.int32, (_N, _HEAD_DIM), 1)
        pad = jnp.where(col == 0, 1.0, 0.0).astype(jnp.bfloat16)
        for h2 in range(_HEADS):
            k = k_all[:, h2 * _HEAD_DIM:(h2 + 1) * _HEAD_DIM]
            kn_s[h2, 0] = jnp.sqrt(jnp.max(jnp.sum(k * k, axis=1)))
            k_s[h2] = k.astype(jnp.bfloat16)
            v = v_all[:, h2 * _HEAD_DIM:(h2 + 1) * _HEAD_DIM].astype(
                jnp.bfloat16
            )
            v_s[h2] = jnp.concatenate([v, pad], axis=1)

    @pl.when(h == 0)
    def _project_q():
        q_all = lax.dot_general(
            q_in_ref[...], wq_ref[...], _DN_T,
            preferred_element_type=jnp.float32,
        ) + bq_ref[...]
        for h2 in range(_HEADS):
            q_s[h2] = q_all[:, h2 * _HEAD_DIM:(h2 + 1) * _HEAD_DIM].astype(
                jnp.bfloat16
            )

    qb = q_s[h]
    qf = qb.astype(jnp.float32)
    qn = jnp.sqrt(jnp.sum(qf * qf, axis=1, keepdims=True))
    m = qn * kn_s[h, 0]

    @pl.when(fast_ref[r] == 1)
    def _windowed():
        st = pl.multiple_of(start_ref[r], _CW)
        logits = lax.dot_general(
            qb, k_s[h, pl.ds(st, _WIN)], _DN_T,
            preferred_element_type=jnp.float32,
        )
        e = jnp.exp(
            logits - m + mask_ref[:, pl.ds(st, _WIN)].astype(jnp.float32)
        ).astype(jnp.bfloat16)
        av_s[...] = lax.dot_general(
            e, v_s[h, pl.ds(st, _WIN)], _DN_N,
            preferred_element_type=jnp.float32,
        )

    @pl.when(fast_ref[r] == 0)
    def _full():
        logits = lax.dot_general(
            qb, k_s[h], _DN_T, preferred_element_type=jnp.float32
        )
        e = jnp.exp(
            logits - m + mask_ref[...].astype(jnp.float32)
        ).astype(jnp.bfloat16)
        av_s[...] = lax.dot_general(
            e, v_s[h], _DN_N, preferred_element_type=jnp.float32
        )

    av = av_s[...]
    attn_s[h] = (
        av[:, :_HEAD_DIM] / av[:, _HEAD_DIM:_HEAD_DIM + 1]
    ).astype(jnp.bfloat16)

    @pl.when(h == _HEADS - 1)
    def _project_out():
        attn_all = jnp.concatenate(
            [attn_s[h2] for h2 in range(_HEADS)], axis=1
        )
        out_ref[...] = lax.dot_general(
            attn_all, wo_ref[...], _DN_T, preferred_element_type=jnp.float32
        ) + bo_ref[...]


def _head_spec():
    return pl.BlockSpec((1, _HEAD_DIM, _EMBED), lambda r, h: (h, 0, 0))


def _bias_spec():
    return pl.BlockSpec((1, 1, _HEAD_DIM), lambda r, h: (h, 0, 0))


_attn = pl.pallas_call(
    _attn_body,
    grid=(_N // _BR, _HEADS),
    in_specs=[
        pl.BlockSpec((_BR, _EMBED), lambda r, h: (r, 0)),
        pl.BlockSpec((_EMBED, _EMBED), lambda r, h: (0, 0)),
        pl.BlockSpec((1, _EMBED), lambda r, h: (0, 0)),
        pl.BlockSpec((_N, _EMBED), lambda r, h: (0, 0)),
        pl.BlockSpec((_EMBED, _EMBED), lambda r, h: (0, 0)),
        pl.BlockSpec((1, _EMBED), lambda r, h: (0, 0)),
        pl.BlockSpec((_N, _EMBED), lambda r, h: (0, 0)),
        pl.BlockSpec((_EMBED, _EMBED), lambda r, h: (0, 0)),
        pl.BlockSpec((1, _EMBED), lambda r, h: (0, 0)),
        pl.BlockSpec((_BR, _N), lambda r, h: (r, 0)),
        pl.BlockSpec(memory_space=pltpu.SMEM),
        pl.BlockSpec(memory_space=pltpu.SMEM),
        pl.BlockSpec((_EMBED, _EMBED), lambda r, h: (0, 0)),
        pl.BlockSpec((1, _EMBED), lambda r, h: (0, 0)),
    ],
    out_specs=pl.BlockSpec((_BR, _EMBED), lambda r, h: (r, 0)),
    out_shape=jax.ShapeDtypeStruct((_N, _EMBED), jnp.float32),
    scratch_shapes=[
        pltpu.VMEM((_HEADS, _N, _HEAD_DIM), jnp.bfloat16),
        pltpu.VMEM((_HEADS, _N, 2 * _HEAD_DIM), jnp.bfloat16),
        pltpu.SMEM((_HEADS, 1), jnp.float32),
        pltpu.VMEM((_HEADS, _BR, _HEAD_DIM), jnp.bfloat16),
        pltpu.VMEM((_HEADS, _BR, _HEAD_DIM), jnp.bfloat16),
        pltpu.VMEM((_BR, 2 * _HEAD_DIM), jnp.float32),
    ],
    compiler_params=pltpu.CompilerParams(
        vmem_limit_bytes=100 * 1024 * 1024,
    ),
)


def kernel(query, key, value, Wq, bq, Wk, bk, Wv, bv, Wo, bo, samples):
    bf = jnp.bfloat16
    mask, cnt = _build_mask(samples)
    nz = cnt.reshape(_N // _BR, _BR // _ROWS_PER_W, _NCHUNK, 16)[..., 0].sum(1)
    total = nz.sum(1, keepdims=True)
    npos = _NCHUNK - _WIN // _CW + 1
    covered = jnp.stack(
        [nz[:, p:p + _WIN // _CW].sum(1) for p in range(npos)], axis=1
    )
    fits = covered == total
    fast = fits.any(1).astype(jnp.int32)
    start = (jnp.argmax(fits, axis=1) * _CW).astype(jnp.int32)
    out = _attn(
        query.astype(bf), (Wq * _SCALE).astype(bf),
        (bq * _SCALE).reshape(1, _EMBED),
        key.astype(bf), Wk.astype(bf), bk.reshape(1, _EMBED),
        value.astype(bf), Wv.astype(bf), bv.reshape(1, _EMBED),
        mask.astype(bf), fast, start, Wo.astype(bf), bo.reshape(1, _EMBED),
    )
    return out.reshape(_N, 1, _EMBED)


# speedup vs baseline: 1.1786x; 1.0014x over previous
import functools
import math

import jax
import jax.numpy as jnp
from jax import lax
from jax.experimental import pallas as pl
from jax.experimental.pallas import tpu as pltpu
from jax.experimental.pallas import tpu_sc as plsc

_N = 2048
_EMBED = 768
_HEADS = 12
_HEAD_DIM = _EMBED // _HEADS
_NUM_SAMPLES = 256
_SCALE = 1.0 / math.sqrt(float(_N))
_NEG = -1e30

_NW = 32
_ROWS_PER_W = _N // _NW
_CHUNK = 16
_G = _NUM_SAMPLES // 16
_NCHUNK = 8
_CW = _N // _NCHUNK
_CHUNK_SHIFT = _CW.bit_length() - 1
_WIN = 4 * _CW


def _mask_body(samples_ref, mask_ref, cnt_ref, idx_v, buf_v, cnt_v, dma_sems):
    wid = lax.axis_index("s") * 2 + lax.axis_index("c")
    base = wid * _ROWS_PER_W
    pltpu.sync_copy(samples_ref.at[pl.ds(base, _ROWS_PER_W)], idx_v)
    neg = jnp.full((16,), _NEG, jnp.float32)
    zero = jnp.zeros((16,), jnp.float32)

    def count(g, accs):
        col = lax.shift_right_logical(
            idx_v[g // _G, pl.ds((g % _G) * 16, 16)], _CHUNK_SHIFT
        )
        return tuple(
            acc + plsc.all_reduce_population_count(col == cc)
            for cc, acc in enumerate(accs)
        )

    accs = lax.fori_loop(
        0,
        _ROWS_PER_W * _NUM_SAMPLES // 16,
        count,
        tuple(jnp.zeros((16,), jnp.int32) for _ in range(_NCHUNK)),
    )
    for cc in range(_NCHUNK):
        cnt_v[pl.ds(cc * 16, 16)] = accs[cc]
    pltpu.sync_copy(cnt_v, cnt_ref.at[wid])

    def fill(i, carry):
        for g in range(_N // 16):
            buf_v[0, i, pl.ds(g * 16, 16)] = neg
            buf_v[1, i, pl.ds(g * 16, 16)] = neg
        return carry

    lax.fori_loop(0, _CHUNK, fill, 0)

    def scatter_chunk(c, b, val):
        def scatter_row(r, carry):
            row = jnp.full((16,), 0, jnp.int32) + r
            for g in range(_G):
                idx = idx_v[c * _CHUNK + r, pl.ds(g * 16, 16)]
                plsc.store_scatter(buf_v.at[b], [row, idx], val)
            return carry

        lax.fori_loop(0, _CHUNK, scatter_row, 0)

    n_chunks = _ROWS_PER_W // _CHUNK
    handles = [None, None]
    for c in range(n_chunks):
        b = c % 2
        if handles[b] is not None:
            handles[b].wait()
            scatter_chunk(c - 2, b, neg)
        scatter_chunk(c, b, zero)
        handles[b] = pltpu.make_async_copy(
            buf_v.at[b],
            mask_ref.at[pl.ds(base + c * _CHUNK, _CHUNK)],
            dma_sems.at[b],
        )
        handles[b].start()
    handles[0].wait()
    handles[1].wait()


@functools.cache
def _get_mask_builder():
    return pl.kernel(
        _mask_body,
        out_type=[
            jax.ShapeDtypeStruct((_N, _N), jnp.float32),
            jax.ShapeDtypeStruct((_NW, _NCHUNK * 16), jnp.int32),
        ],
        mesh=plsc.VectorSubcoreMesh(core_axis_name="c", subcore_axis_name="s"),
        scratch_types=[
            pltpu.VMEM((_ROWS_PER_W, _NUM_SAMPLES), jnp.int32),
            pltpu.VMEM((2, _CHUNK, _N), jnp.float32),
            pltpu.VMEM((_NCHUNK * 16,), jnp.int32),
            pltpu.SemaphoreType.DMA((2,)),
        ],
        compiler_params=pltpu.CompilerParams(
            needs_layout_passes=False, use_tc_tiling_on_sc=False
        ),
    )


def _build_mask(samples):
    return _get_mask_builder()(samples)


_BR = 512
_NSPLIT = 2
_DN_T = (((1,), (1,)), ((), ()))
_DN_N = (((1,), (0,)), ((), ()))


def _attn_body(q_in_ref, wq_ref, bq_ref, key_ref, wk_ref, bk_ref,
               value_ref, wv_ref, bv_ref, mask_ref, fast_ref, start_ref,
               wo_ref, bo_ref, out_ref, k_s, v_s, kn_s, q_s, attn_s, av_s):
    r = pl.program_id(0)
    h = pl.program_id(1)

    @pl.when(jnp.logical_and(r == 0, h == 0))
    def _project_kv():
        k_all = lax.dot_general(
            key_ref[...], wk_ref[...], _DN_T,
            preferred_element_type=jnp.float32,
        ) + bk_ref[...]
        v_all = lax.dot_general(
            value_ref[...], wv_ref[...], _DN_T,
            preferred_element_type=jnp.float32,
        ) + bv_ref[...]
        col = lax.broadcasted_iota(jnp.int32, (_N, _HEAD_DIM), 1)
        pad = jnp.where(col == 0, 1.0, 0.0).astype(jnp.bfloat16)
        for h2 in range(_HEADS):
            k = k_all[:, h2 * _HEAD_DIM:(h2 + 1) * _HEAD_DIM]
            kn_s[h2, 0] = jnp.sqrt(jnp.max(jnp.sum(k * k, axis=1)))
            k_s[h2] = k.astype(jnp.bfloat16)
            v = v_all[:, h2 * _HEAD_DIM:(h2 + 1) * _HEAD_DIM].astype(
                jnp.bfloat16
            )
            v_s[h2] = jnp.concatenate([v, pad], axis=1)

    @pl.when(h == 0)
    def _project_q():
        q_all = lax.dot_general(
            q_in_ref[...], wq_ref[...], _DN_T,
            preferred_element_type=jnp.float32,
        ) + bq_ref[...]
        for h2 in range(_HEADS):
            q_s[h2] = q_all[:, h2 * _HEAD_DIM:(h2 + 1) * _HEAD_DIM].astype(
                jnp.bfloat16
            )

    qb = q_s[h]
    qf = qb.astype(jnp.float32)
    qn = jnp.sqrt(jnp.sum(qf * qf, axis=1, keepdims=True))
    m = qn * kn_s[h, 0]

    @pl.when(fast_ref[r] == 1)
    def _windowed():
        st = pl.multiple_of(start_ref[r], _CW)
        logits = lax.dot_general(
            qb, k_s[h, pl.ds(st, _WIN)], _DN_T,
            preferred_element_type=jnp.float32,
        )
        e = jnp.exp(
            logits - m + mask_ref[:, pl.ds(st, _WIN)].astype(jnp.float32)
        ).astype(jnp.bfloat16)
        av_s[...] = lax.dot_general(
            e, v_s[h, pl.ds(st, _WIN)], _DN_N,
            preferred_element_type=jnp.float32,
        )

    @pl.when(fast_ref[r] == 0)
    def _full():
        logits = lax.dot_general(
            qb, k_s[h], _DN_T, preferred_element_type=jnp.float32
        )
        e = jnp.exp(
            logits - m + mask_ref[...].astype(jnp.float32)
        ).astype(jnp.bfloat16)
        av_s[...] = lax.dot_general(
            e, v_s[h], _DN_N, preferred_element_type=jnp.float32
        )

    av = av_s[...]
    attn_s[h] = (
        av[:, :_HEAD_DIM] / av[:, _HEAD_DIM:_HEAD_DIM + 1]
    ).astype(jnp.bfloat16)

    @pl.when(h == _HEADS - 1)
    def _project_out():
        attn_all = jnp.concatenate(
            [attn_s[h2] for h2 in range(_HEADS)], axis=1
        )
        out_ref[...] = lax.dot_general(
            attn_all, wo_ref[...], _DN_T, preferred_element_type=jnp.float32
        ) + bo_ref[...]


def _head_spec():
    return pl.BlockSpec((1, _HEAD_DIM, _EMBED), lambda r, h: (h, 0, 0))


def _bias_spec():
    return pl.BlockSpec((1, 1, _HEAD_DIM), lambda r, h: (h, 0, 0))


_attn = pl.pallas_call(
    _attn_body,
    grid=(_N // _BR, _HEADS),
    in_specs=[
        pl.BlockSpec((_BR, _EMBED), lambda r, h: (r, 0)),
        pl.BlockSpec((_EMBED, _EMBED), lambda r, h: (0, 0)),
        pl.BlockSpec((1, _EMBED), lambda r, h: (0, 0)),
        pl.BlockSpec((_N, _EMBED), lambda r, h: (0, 0)),
        pl.BlockSpec((_EMBED, _EMBED), lambda r, h: (0, 0)),
        pl.BlockSpec((1, _EMBED), lambda r, h: (0, 0)),
        pl.BlockSpec((_N, _EMBED), lambda r, h: (0, 0)),
        pl.BlockSpec((_EMBED, _EMBED), lambda r, h: (0, 0)),
        pl.BlockSpec((1, _EMBED), lambda r, h: (0, 0)),
        pl.BlockSpec((_BR, _N), lambda r, h: (r, 0)),
        pl.BlockSpec(memory_space=pltpu.SMEM),
        pl.BlockSpec(memory_space=pltpu.SMEM),
        pl.BlockSpec((_EMBED, _EMBED), lambda r, h: (0, 0)),
        pl.BlockSpec((1, _EMBED), lambda r, h: (0, 0)),
    ],
    out_specs=pl.BlockSpec((_BR, _EMBED), lambda r, h: (r, 0)),
    out_shape=jax.ShapeDtypeStruct((_N, _EMBED), jnp.float32),
    scratch_shapes=[
        pltpu.VMEM((_HEADS, _N, _HEAD_DIM), jnp.bfloat16),
        pltpu.VMEM((_HEADS, _N, 2 * _HEAD_DIM), jnp.bfloat16),
        pltpu.SMEM((_HEADS, 1), jnp.float32),
        pltpu.VMEM((_HEADS, _BR, _HEAD_DIM), jnp.bfloat16),
        pltpu.VMEM((_HEADS, _BR, _HEAD_DIM), jnp.bfloat16),
        pltpu.VMEM((_BR, 2 * _HEAD_DIM), jnp.float32),
    ],
    compiler_params=pltpu.CompilerParams(
        vmem_limit_bytes=100 * 1024 * 1024,
    ),
)


def kernel(query, key, value, Wq, bq, Wk, bk, Wv, bv, Wo, bo, samples):
    bf = jnp.bfloat16
    mask, cnt = _build_mask(samples)
    nz = cnt.reshape(_N // _BR, _BR // _ROWS_PER_W, _NCHUNK, 16)[..., 0].sum(1)
    total = nz.sum(1, keepdims=True)
    npos = _NCHUNK - _WIN // _CW + 1
    covered = jnp.stack(
        [nz[:, p:p + _WIN // _CW].sum(1) for p in range(npos)], axis=1
    )
    fits = covered == total
    fast = fits.any(1).astype(jnp.int32)
    start = (jnp.argmax(fits, axis=1) * _CW).astype(jnp.int32)
    out = _attn(
        query.astype(bf), (Wq * _SCALE).astype(bf),
        (bq * _SCALE).reshape(1, _EMBED),
        key.astype(bf), Wk.astype(bf), bk.reshape(1, _EMBED),
        value.astype(bf), Wv.astype(bf), bv.reshape(1, _EMBED),
        mask.astype(bf), fast, start, Wo.astype(bf), bo.reshape(1, _EMBED),
    )
    return out.reshape(_N, 1, _EMBED)
